# baseline (speedup 1.0000x reference)
import functools

import jax
import jax.numpy as jnp
from jax import lax
from jax.experimental import pallas as pl
from jax.experimental.pallas import tpu as pltpu
from jax.experimental.pallas import tpu_sc as plsc

BATCH = 16
SEQ = 512
IN_SIZE = 256
PROJ = 256
HID = 128
DENSE = 256
NUM_STEPS = 10
BETA = 0.95
THRESH = 1.0

N_NODES = BATCH * SEQ
FEAT = SEQ * HID
N_CHUNKS = 8
CHUNK = FEAT // N_CHUNKS

NW = 32
ROWS = N_NODES // NW
LANES = 16
NJ = HID // LANES


def _xw_kernel(x_ref, wp_ref, bp_ref, wg_ref, xw_ref):
    xb = x_ref[...].reshape(4 * SEQ, IN_SIZE)
    xp = jax.lax.dot_general(
        xb, wp_ref[...],
        dimension_numbers=(((1,), (1,)), ((), ())),
        preferred_element_type=jnp.float32) + bp_ref[...]
    xw_ref[...] = jax.lax.dot_general(
        xp, wg_ref[...],
        dimension_numbers=(((1,), (1,)), ((), ())),
        preferred_element_type=jnp.float32)


def _sc_agg(xw_hbm, cl_hbm, cr_hbm, cs_hbm, bg_hbm, cur_hbm,
            ext_v, cl_v, cr_v, cs_v, bg_v):
    wid = lax.axis_index("s") * 2 + lax.axis_index("c")
    base = wid * ROWS
    t0 = (wid % 2) * ROWS

    pltpu.sync_copy(cl_hbm.at[pl.ds(t0, ROWS)], cl_v)
    pltpu.sync_copy(cr_hbm.at[pl.ds(t0, ROWS)], cr_v)
    pltpu.sync_copy(cs_hbm.at[pl.ds(t0, ROWS)], cs_v)
    pltpu.sync_copy(bg_hbm, bg_v)

    HALF = ROWS // 2
    for h in range(2):
        hbase = base + h * HALF
        pltpu.sync_copy(xw_hbm.at[pl.ds(hbase, HALF)], ext_v.at[pl.ds(1, HALF)])
        lrow = jnp.maximum(hbase - 1, 0)
        pltpu.sync_copy(xw_hbm.at[pl.ds(lrow, 1)], ext_v.at[pl.ds(0, 1)])
        rrow = jnp.minimum(hbase + HALF, N_NODES - 1)
        pltpu.sync_copy(xw_hbm.at[pl.ds(rrow, 1)], ext_v.at[pl.ds(HALF + 1, 1)])

        def body(i, carry):
            r = HALF - 1 - i
            tr = h * HALF + r
            clv = cl_v[tr]
            crv = cr_v[tr]
            csv = cs_v[tr]
            for j in range(NJ):
                sl = pl.ds(j * LANES, LANES)
                l = ext_v[r, sl]
                s = ext_v[r + 1, sl]
                rt = ext_v[r + 2, sl]
                ext_v[r + 2, sl] = ((clv * l + crv * rt) + csv * s) + bg_v[j]
            return carry

        lax.fori_loop(0, HALF, body, 0)
        pltpu.sync_copy(ext_v.at[pl.ds(2, HALF)], cur_hbm.at[pl.ds(hbase, HALF)])


def _dense_kernel(cur_ref, wd_ref, bd_ref, wo_ref, bo_ref, out_ref, acc_ref):
    k = pl.program_id(0)

    @pl.when(k == 0)
    def _():
        acc_ref[...] = jnp.zeros_like(acc_ref)

    cur = cur_ref[...]
    mem = cur
    for _ in range(NUM_STEPS - 1):
        reset = jnp.where(mem > THRESH, jnp.float32(THRESH), jnp.float32(0.0))
        mem = BETA * mem + cur - reset

    acc_ref[...] += jax.lax.dot_general(
        mem, wd_ref[...],
        dimension_numbers=(((1,), (1,)), ((), ())),
        preferred_element_type=jnp.float32)

    @pl.when(k == N_CHUNKS - 1)
    def _():
        y = jnp.maximum(acc_ref[...] + bd_ref[...], 0.0)
        o = jnp.sum(y * wo_ref[...], axis=1, keepdims=True)
        out_ref[...] = jax.nn.sigmoid(o + bo_ref[0, 0])


def kernel(x, W_proj, b_proj, W_gcn, b_gcn, W_dense, b_dense, W_out, b_out):
    bp2 = b_proj.reshape(1, PROJ)
    bd2 = b_dense.reshape(1, DENSE)
    bo2 = b_out.reshape(1, 1)

    t = jnp.arange(SEQ)
    end = (t == 0) | (t == SEQ - 1)
    dinv = lax.rsqrt(jnp.where(end, 2.0, 3.0))
    dinv_m1 = lax.rsqrt(jnp.where((t == 1) | (t == 0), 2.0, 3.0))
    dinv_p1 = lax.rsqrt(jnp.where((t == SEQ - 2) | (t == SEQ - 1), 2.0, 3.0))
    cl = jnp.where(t == 0, 0.0, dinv_m1 * dinv)
    cr = jnp.where(t == SEQ - 1, 0.0, dinv_p1 * dinv)
    cs = dinv * dinv
    cl_t = jnp.broadcast_to(cl[:, None], (SEQ, LANES))
    cr_t = jnp.broadcast_to(cr[:, None], (SEQ, LANES))
    cs_t = jnp.broadcast_to(cs[:, None], (SEQ, LANES))
    bg_t = b_gcn.reshape(NJ, LANES)

    xw = pl.pallas_call(
        _xw_kernel,
        grid=(BATCH // 4,),
        in_specs=[
            pl.BlockSpec((4, SEQ, IN_SIZE), lambda b: (b, 0, 0)),
            pl.BlockSpec((PROJ, IN_SIZE), lambda b: (0, 0)),
            pl.BlockSpec((1, PROJ), lambda b: (0, 0)),
            pl.BlockSpec((HID, PROJ), lambda b: (0, 0)),
        ],
        out_specs=pl.BlockSpec((4 * SEQ, HID), lambda b: (b, 0)),
        out_shape=jax.ShapeDtypeStruct((N_NODES, HID), jnp.float32),
    )(x, W_proj, bp2, W_gcn)

    sc_agg = functools.partial(
        pl.kernel, _sc_agg,
        mesh=plsc.VectorSubcoreMesh(core_axis_name="c", subcore_axis_name="s"),
        out_type=jax.ShapeDtypeStruct((N_NODES, HID), jnp.float32),
        scratch_types=[
            pltpu.VMEM((ROWS // 2 + 2, HID), jnp.float32),
            pltpu.VMEM((ROWS, LANES), jnp.float32),
            pltpu.VMEM((ROWS, LANES), jnp.float32),
            pltpu.VMEM((ROWS, LANES), jnp.float32),
            pltpu.VMEM((NJ, LANES), jnp.float32),
        ],
    )()
    cur = sc_agg(xw, cl_t, cr_t, cs_t, bg_t)

    cur2 = cur.reshape(BATCH, FEAT)

    out = pl.pallas_call(
        _dense_kernel,
        grid=(N_CHUNKS,),
        in_specs=[
            pl.BlockSpec((BATCH, CHUNK), lambda k: (0, k)),
            pl.BlockSpec((DENSE, CHUNK), lambda k: (0, k)),
            pl.BlockSpec((1, DENSE), lambda k: (0, 0)),
            pl.BlockSpec((1, DENSE), lambda k: (0, 0)),
            pl.BlockSpec(memory_space=pltpu.SMEM),
        ],
        out_specs=pl.BlockSpec((BATCH, 1), lambda k: (0, 0)),
        out_shape=jax.ShapeDtypeStruct((BATCH, 1), jnp.float32),
        scratch_shapes=[pltpu.VMEM((BATCH, DENSE), jnp.float32)],
    )(cur2, W_dense, bd2, W_out, bo2)

    return out


# speedup vs baseline: 2.6983x; 2.6983x over previous
import jax
import jax.numpy as jnp
from jax.experimental import pallas as pl
from jax.experimental.pallas import tpu as pltpu

BATCH = 16
SEQ = 512
IN_SIZE = 256
PROJ = 256
HID = 128
DENSE = 256
NUM_STEPS = 10
BETA = 0.95
THRESH = 1.0

N_CHUNKS = 4
SBLK = SEQ // N_CHUNKS


def _two_matmul(v, wp_ref, bp_ref, wg_ref):
    xp = jax.lax.dot_general(
        v, wp_ref[...],
        dimension_numbers=(((1,), (1,)), ((), ())),
        preferred_element_type=jnp.float32) + bp_ref[...]
    return jax.lax.dot_general(
        xp, wg_ref[...],
        dimension_numbers=(((1,), (1,)), ((), ())),
        preferred_element_type=jnp.float32)


def _fused_kernel(xs_ref, xlo_ref, xhi_ref, wp_ref, bp_ref, wg_ref, bg_ref,
                  wd_ref, bd_ref, wo_ref, bo_ref, out_ref, acc_ref):
    k = pl.program_id(0)

    @pl.when(k == 0)
    def _():
        acc_ref[...] = jnp.zeros_like(acc_ref)

    base = k * SBLK
    xs2 = xs_ref[...].reshape(BATCH * SBLK, IN_SIZE)
    xw = _two_matmul(xs2, wp_ref, bp_ref, wg_ref)
    xw3 = xw.reshape(BATCH, SBLK, HID)
    xlw = _two_matmul(xlo_ref[:, 7, :], wp_ref, bp_ref, wg_ref)
    xrw = _two_matmul(xhi_ref[:, 0, :], wp_ref, bp_ref, wg_ref)

    xw_prev = jnp.concatenate([xlw[:, None, :], xw3[:, :-1, :]], axis=1)
    xw_next = jnp.concatenate([xw3[:, 1:, :], xrw[:, None, :]], axis=1)

    t = base + jax.lax.broadcasted_iota(jnp.int32, (1, SBLK, 1), 1)
    first = t == 0
    last = t == SEQ - 1
    dinv = jax.lax.rsqrt(jnp.where(first | last, 2.0, 3.0))
    dinv_m1 = jax.lax.rsqrt(jnp.where((t == 1) | first, 2.0, 3.0))
    dinv_p1 = jax.lax.rsqrt(jnp.where((t == SEQ - 2) | last, 2.0, 3.0))
    cl = jnp.where(first, 0.0, dinv_m1 * dinv)
    cr = jnp.where(last, 0.0, dinv_p1 * dinv)
    cs = dinv * dinv

    cur = ((cl * xw_prev + cr * xw_next) + cs * xw3) + bg_ref[...].reshape(1, 1, HID)

    mem = cur
    for _ in range(NUM_STEPS - 1):
        reset = jnp.where(mem > THRESH, jnp.float32(THRESH), jnp.float32(0.0))
        mem = BETA * mem + cur - reset

    acc_ref[...] += jax.lax.dot_general(
        mem.reshape(BATCH, SBLK * HID), wd_ref[...],
        dimension_numbers=(((1,), (1,)), ((), ())),
        preferred_element_type=jnp.float32)

    @pl.when(k == N_CHUNKS - 1)
    def _():
        y = jnp.maximum(acc_ref[...] + bd_ref[...], 0.0)
        o = jnp.sum(y * wo_ref[...], axis=1, keepdims=True)
        out_ref[...] = jax.nn.sigmoid(o + bo_ref[0, 0])


def kernel(x, W_proj, b_proj, W_gcn, b_gcn, W_dense, b_dense, W_out, b_out):
    bp2 = b_proj.reshape(1, PROJ)
    bg2 = b_gcn.reshape(1, HID)
    bd2 = b_dense.reshape(1, DENSE)
    bo2 = b_out.reshape(1, 1)


    out = pl.pallas_call(
        _fused_kernel,
        grid=(N_CHUNKS,),
        in_specs=[
            pl.BlockSpec((BATCH, SBLK, IN_SIZE), lambda k: (0, k, 0)),
            pl.BlockSpec((BATCH, 8, IN_SIZE),
                         lambda k: (0, jnp.maximum(k * (SBLK // 8) - 1, 0), 0)),
            pl.BlockSpec((BATCH, 8, IN_SIZE),
                         lambda k: (0, jnp.minimum(k * (SBLK // 8) + SBLK // 8,
                                                   SEQ // 8 - 1), 0)),
            pl.BlockSpec((PROJ, IN_SIZE), lambda k: (0, 0)),
            pl.BlockSpec((1, PROJ), lambda k: (0, 0)),
            pl.BlockSpec((HID, PROJ), lambda k: (0, 0)),
            pl.BlockSpec((1, HID), lambda k: (0, 0)),
            pl.BlockSpec((DENSE, SBLK * HID), lambda k: (0, k)),
            pl.BlockSpec((1, DENSE), lambda k: (0, 0)),
            pl.BlockSpec((1, DENSE), lambda k: (0, 0)),
            pl.BlockSpec(memory_space=pltpu.SMEM),
        ],
        out_specs=pl.BlockSpec((BATCH, 1), lambda k: (0, 0)),
        out_shape=jax.ShapeDtypeStruct((BATCH, 1), jnp.float32),
        scratch_shapes=[pltpu.VMEM((BATCH, DENSE), jnp.float32)],
    )(x, x, x, W_proj, bp2, W_gcn, bg2, W_dense, bd2, W_out, bo2)

    return out
